# baseline (speedup 1.0000x reference)
import functools

import jax
import jax.numpy as jnp
from jax import lax
from jax.experimental import pallas as pl
from jax.experimental.pallas import tpu as pltpu

_EPS = float(jnp.finfo(jnp.float32).eps)


def _mish(x):
    sp = jnp.maximum(x, 0.0) + jnp.log(1.0 + jnp.exp(-jnp.abs(x)))
    return x * jnp.tanh(sp)


def _rmsnorm(x, w):
    ms = jnp.mean(x * x, axis=-1, keepdims=True)
    return x * lax.rsqrt(ms + _EPS) * w


def _dw5(pad_ref, xn, w, H, W, C):
    pad_ref[...] = jnp.zeros((H + 4, W + 4, C), jnp.float32)
    pad_ref[2:2 + H, 2:2 + W, :] = xn
    acc = jnp.zeros((H, W, C), jnp.float32)
    for kh in range(5):
        for kw in range(5):
            acc += pad_ref[kh:kh + H, kw:kw + W, :] * w[kh, kw]
    return acc


def _fused_kernel(x_ref, rn1_ref, os1_ref, wqkv_ref, bqkv_ref,
                  w3_ref, b3_ref, wp_ref, bp_ref,
                  g1_ref, rn2_ref, os2_ref,
                  fwk_ref, fwv_ref, fwr_ref, g2_ref,
                  w1s_ref, w1x_ref, b1_ref, w2_ref, b2_ref,
                  w11s_ref, w11x_ref, b11_ref,
                  o_ref,
                  pad_ref, bpad_ref, vpad_ref, yimg_ref,
                  *, H, W, ws, scale):
    C = o_ref.shape[-1]
    nWh, nWw = H // ws, W // ws
    nW = nWh * nWw
    NN = ws * ws
    bf = jnp.bfloat16

    x = x_ref[...].astype(jnp.float32)

    xn = _rmsnorm(x, rn1_ref[...])
    conv1 = _dw5(pad_ref, xn, os1_ref[...], H, W, C)

    wins = []
    for wi in range(nWh):
        for wj in range(nWw):
            wins.append(conv1[wi * ws:(wi + 1) * ws,
                              wj * ws:(wj + 1) * ws, :].reshape(1, NN, C))
    xw = jnp.concatenate(wins, axis=0).reshape(nW * NN, C)

    qkv = jnp.dot(xw.astype(bf), wqkv_ref[...],
                  preferred_element_type=jnp.float32) + bqkv_ref[...]
    q = qkv[:, :C]
    k = qkv[:, C:2 * C]
    v = qkv[:, 2 * C:]

    vpad_ref[...] = jnp.zeros((nW, ws + 2, ws + 2, C), jnp.float32)
    vpad_ref[:, 1:ws + 1, 1:ws + 1, :] = v.reshape(nW, ws, ws, C)
    w3 = w3_ref[...]
    lepe = jnp.zeros((nW, ws, ws, C), jnp.float32)
    for kh in range(3):
        for kw in range(3):
            lepe += vpad_ref[:, kh:kh + ws, kw:kw + ws, :] * w3[kh, kw]
    lepe = lepe.reshape(nW, NN, C) + b3_ref[...].reshape(1, 1, C)

    q3 = q.reshape(nW, NN, C).astype(bf)
    k3 = k.reshape(nW, NN, C).astype(bf)
    v3 = v.reshape(nW, NN, C).astype(bf)
    attn = jnp.einsum('gnc,gmc->gnm', q3, k3,
                      preferred_element_type=jnp.float32) * scale
    attn = jax.nn.softmax(attn, axis=-1).astype(bf)
    yw = jnp.einsum('gnm,gmc->gnc', attn, v3,
                    preferred_element_type=jnp.float32) + lepe
    yw = (jnp.dot(yw.reshape(nW * NN, C).astype(bf), wp_ref[...],
                  preferred_element_type=jnp.float32) + bp_ref[...]).reshape(nW, NN, C)

    for wi in range(nWh):
        for wj in range(nWw):
            yimg_ref[wi * ws:(wi + 1) * ws, wj * ws:(wj + 1) * ws, :] = (
                yw[wi * nWw + wj].reshape(ws, ws, C))
    y = yimg_ref[...]

    x1 = x + g1_ref[...] * y
    xn2 = _rmsnorm(x1, rn2_ref[...])
    conv2 = _dw5(pad_ref, xn2, os2_ref[...], H, W, C)

    a = conv2.reshape(H * W, C).astype(bf)
    kk = jnp.dot(a, fwk_ref[...], preferred_element_type=jnp.float32)
    kk = jnp.square(jnp.maximum(kk, 0.0))
    kv = jnp.dot(kk.astype(bf), fwv_ref[...], preferred_element_type=jnp.float32)
    r = jax.nn.sigmoid(jnp.dot(a, fwr_ref[...], preferred_element_type=jnp.float32))
    y2 = (r * kv).reshape(H, W, C)
    x2 = x1 + g2_ref[...] * y2

    def conv3(inp, w_ref):
        bpad_ref[...] = jnp.zeros((H + 2, W + 2, C), bf)
        bpad_ref[1:1 + H, 1:1 + W, :] = inp.astype(bf)
        acc = jnp.zeros((H * W, C), jnp.float32)
        for kh in range(3):
            for kw in range(3):
                acc += jnp.dot(
                    bpad_ref[kh:kh + H, kw:kw + W, :].reshape(H * W, C),
                    w_ref[kh, kw], preferred_element_type=jnp.float32)
        return acc

    h = conv3(x, w1s_ref) + conv3(x2, w1x_ref) + b1_ref[...]
    h = _mish(h)
    h = conv3(h.reshape(H, W, C), w2_ref) + b2_ref[...]
    h = _mish(h)
    o2 = (jnp.dot(x.reshape(H * W, C).astype(bf), w11s_ref[...],
                  preferred_element_type=jnp.float32)
          + jnp.dot(x2.reshape(H * W, C).astype(bf), w11x_ref[...],
                    preferred_element_type=jnp.float32)
          + b11_ref[...])
    o_ref[...] = (h + o2).reshape(H, W, C).astype(o_ref.dtype)


def kernel(x, rn1_w, rn2_w, gamma1, gamma2, att_os_w, ffn_os_w,
           att_wq, att_wk, att_wv, att_bq, att_bk, att_bv, att_w3, att_b3,
           att_wp, att_bp, ffn_wk, ffn_wv, ffn_wr,
           conv_w1, conv_b1, conv_w2, conv_b2, conv_w11, conv_b11):
    B, N, C = x.shape
    H = W = 32
    ws = 8
    nW = (H // ws) * (W // ws)
    scale = C ** (-0.5)
    bf = jnp.bfloat16

    img = x.reshape(B, H, W, C)
    wqkv = jnp.concatenate([att_wq, att_wk, att_wv], axis=1).astype(bf)
    bqkv = jnp.concatenate([att_bq, att_bk, att_bv], axis=1)
    w1s = conv_w1[:, :, :C, :].astype(bf)
    w1x = conv_w1[:, :, C:, :].astype(bf)
    w11s = conv_w11[:C, :].astype(bf)
    w11x = conv_w11[C:, :].astype(bf)

    kern = functools.partial(_fused_kernel, H=H, W=W, ws=ws, scale=scale)

    def full(shape):
        n = len(shape)
        return pl.BlockSpec(shape, lambda b, _n=n: (0,) * _n)

    return pl.pallas_call(
        kern,
        out_shape=jax.ShapeDtypeStruct((B, H, W, C), img.dtype),
        grid=(B,),
        in_specs=[pl.BlockSpec((None, H, W, C), lambda b: (b, 0, 0, 0)),
                  full((1, 1, C)), full((5, 5, C)),
                  full((C, 3 * C)), full((1, 3 * C)),
                  full((3, 3, C)), full((1, 1, C)),
                  full((C, C)), full((1, C)),
                  full((1, 1, C)), full((1, 1, C)), full((5, 5, C)),
                  full((C, 4 * C)), full((4 * C, C)), full((C, C)),
                  full((1, 1, C)),
                  full((3, 3, C, C)), full((3, 3, C, C)), full((1, C)),
                  full((3, 3, C, C)), full((1, C)),
                  full((C, C)), full((C, C)), full((1, C))],
        out_specs=pl.BlockSpec((None, H, W, C), lambda b: (b, 0, 0, 0)),
        scratch_shapes=[pltpu.VMEM((H + 4, W + 4, C), jnp.float32),
                        pltpu.VMEM((H + 2, W + 2, C), bf),
                        pltpu.VMEM((nW, ws + 2, ws + 2, C), jnp.float32),
                        pltpu.VMEM((H, W, C), jnp.float32)],
        compiler_params=pltpu.CompilerParams(dimension_semantics=("parallel",)),
    )(img,
      rn1_w.reshape(1, 1, C), att_os_w,
      wqkv, bqkv,
      att_w3, att_b3,
      att_wp.astype(bf), att_bp,
      gamma1.reshape(1, 1, C), rn2_w.reshape(1, 1, C), ffn_os_w,
      ffn_wk.astype(bf), ffn_wv.astype(bf), ffn_wr.astype(bf),
      gamma2.reshape(1, 1, C),
      w1s, w1x, conv_b1.reshape(1, C), conv_w2.astype(bf),
      conv_b2.reshape(1, C),
      w11s, w11x, conv_b11.reshape(1, C)).reshape(B, N, C)


# speedup vs baseline: 1.3279x; 1.3279x over previous
import functools

import jax
import jax.numpy as jnp
from jax import lax
from jax.experimental import pallas as pl
from jax.experimental.pallas import tpu as pltpu

_EPS = float(jnp.finfo(jnp.float32).eps)


def _mish(x):
    sp = jnp.maximum(x, 0.0) + jnp.log(1.0 + jnp.exp(-jnp.abs(x)))
    return x * jnp.tanh(sp)


def _rmsnorm(x, w):
    ms = jnp.mean(x * x, axis=-1, keepdims=True)
    return x * lax.rsqrt(ms + _EPS) * w


def _dw5(pad_ref, xn, w, H, W, C):
    pad_ref[...] = jnp.zeros((H + 4, W + 4, C), jnp.float32)
    pad_ref[2:2 + H, 2:2 + W, :] = xn
    acc = jnp.zeros((H, W, C), jnp.float32)
    for kh in range(5):
        for kw in range(5):
            acc += pad_ref[kh:kh + H, kw:kw + W, :] * w[kh, kw]
    return acc


def _fused_kernel(x_ref, rn1_ref, os1_ref, wqkv_ref, bqkv_ref,
                  w3_ref, b3_ref, wp_ref, bp_ref,
                  g1_ref, rn2_ref, os2_ref,
                  fkr_ref, fwv_ref, g2_ref,
                  w1c_ref, b1_ref, w2c_ref, b2_ref, w11_ref, b11_ref,
                  o_ref,
                  pad_ref, vpad_ref, yimg_ref, ypad_ref,
                  *, H, W, ws, scale, Hd):
    C = o_ref.shape[-1]
    N = H * W
    nWh, nWw = H // ws, W // ws
    nW = nWh * nWw
    NN = ws * ws
    P = 64

    xf = x_ref[...].astype(jnp.float32)

    xn = _rmsnorm(xf, rn1_ref[...])
    conv1 = _dw5(pad_ref, xn.reshape(H, W, C), os1_ref[...], H, W, C)

    wins = []
    for wi in range(nWh):
        for wj in range(nWw):
            wins.append(conv1[wi * ws:(wi + 1) * ws,
                              wj * ws:(wj + 1) * ws, :].reshape(1, NN, C))
    xw = jnp.concatenate(wins, axis=0).reshape(nW * NN, C)

    qkv = jnp.dot(xw, wqkv_ref[...],
                  preferred_element_type=jnp.float32) + bqkv_ref[...]
    q = qkv[:, :C]
    k = qkv[:, C:2 * C]
    v = qkv[:, 2 * C:]

    vpad_ref[...] = jnp.zeros((nW, ws + 2, ws + 2, C), jnp.float32)
    vpad_ref[:, 1:ws + 1, 1:ws + 1, :] = v.reshape(nW, ws, ws, C)
    w3 = w3_ref[...]
    lepe = jnp.zeros((nW, ws, ws, C), jnp.float32)
    for kh in range(3):
        for kw in range(3):
            lepe += vpad_ref[:, kh:kh + ws, kw:kw + ws, :] * w3[kh, kw]
    lepe = lepe.reshape(nW, NN, C) + b3_ref[...].reshape(1, 1, C)

    q3 = q.reshape(nW, NN, C)
    k3 = k.reshape(nW, NN, C)
    v3 = v.reshape(nW, NN, C)
    attn = jnp.einsum('gnc,gmc->gnm', q3, k3,
                      preferred_element_type=jnp.float32) * scale
    attn = jax.nn.softmax(attn, axis=-1)
    yw = jnp.einsum('gnm,gmc->gnc', attn, v3,
                    preferred_element_type=jnp.float32) + lepe
    yw = (jnp.dot(yw.reshape(nW * NN, C), wp_ref[...],
                  preferred_element_type=jnp.float32) + bp_ref[...]).reshape(nW, NN, C)

    for wi in range(nWh):
        for wj in range(nWw):
            yimg_ref[wi * ws:(wi + 1) * ws, wj * ws:(wj + 1) * ws, :] = (
                yw[wi * nWw + wj].reshape(ws, ws, C))
    y = yimg_ref[...].reshape(N, C)

    x1 = xf + g1_ref[...] * y
    xn2 = _rmsnorm(x1, rn2_ref[...])
    conv2 = _dw5(pad_ref, xn2.reshape(H, W, C), os2_ref[...], H, W, C)

    a = conv2.reshape(N, C)
    kr = jnp.dot(a, fkr_ref[...], preferred_element_type=jnp.float32)
    kk = jnp.square(jnp.maximum(kr[:, :Hd], 0.0))
    r = jax.nn.sigmoid(kr[:, Hd:])
    kv = jnp.dot(kk, fwv_ref[...], preferred_element_type=jnp.float32)
    x2 = x1 + g2_ref[...] * (r * kv)

    wcol = lax.broadcasted_iota(jnp.int32, (N, C), 0) % W
    mask_l = wcol == 0
    mask_r = wcol == W - 1

    ypad_ref[0:P, :] = jnp.zeros((P, 9 * C), jnp.float32)
    ypad_ref[P + N:P + N + P, :] = jnp.zeros((P, 9 * C), jnp.float32)

    def conv3w(inp, wc_ref):
        yy = jnp.dot(inp, wc_ref[...], preferred_element_type=jnp.float32)
        ypad_ref[P:P + N, :] = yy
        acc = jnp.zeros((N, C), jnp.float32)
        for kh in range(3):
            for kw in range(3):
                t = kh * 3 + kw
                off = W * (kh - 1) + (kw - 1)
                sl = ypad_ref[P + off:P + off + N, t * C:(t + 1) * C]
                if kw == 0:
                    sl = jnp.where(mask_l, 0.0, sl)
                elif kw == 2:
                    sl = jnp.where(mask_r, 0.0, sl)
                acc += sl
        return acc

    cc = jnp.concatenate([xf, x2], axis=1)
    h = _mish(conv3w(cc, w1c_ref) + b1_ref[...])
    h = _mish(conv3w(h, w2c_ref) + b2_ref[...])
    o2 = jnp.dot(cc, w11_ref[...], preferred_element_type=jnp.float32) + b11_ref[...]
    o_ref[...] = (h + o2).astype(o_ref.dtype)


def kernel(x, rn1_w, rn2_w, gamma1, gamma2, att_os_w, ffn_os_w,
           att_wq, att_wk, att_wv, att_bq, att_bk, att_bv, att_w3, att_b3,
           att_wp, att_bp, ffn_wk, ffn_wv, ffn_wr,
           conv_w1, conv_b1, conv_w2, conv_b2, conv_w11, conv_b11):
    B, N, C = x.shape
    H = W = 32
    ws = 8
    nW = (H // ws) * (W // ws)
    Hd = ffn_wk.shape[1]
    scale = C ** (-0.5)

    wqkv = jnp.concatenate([att_wq, att_wk, att_wv], axis=1)
    bqkv = jnp.concatenate([att_bq, att_bk, att_bv], axis=1)
    fkr = jnp.concatenate([ffn_wk, ffn_wr], axis=1)
    w1c = conv_w1.reshape(9, 2 * C, C).transpose(1, 0, 2).reshape(2 * C, 9 * C)
    w2c = conv_w2.reshape(9, C, C).transpose(1, 0, 2).reshape(C, 9 * C)

    kern = functools.partial(_fused_kernel, H=H, W=W, ws=ws, scale=scale, Hd=Hd)

    def full(shape):
        n = len(shape)
        return pl.BlockSpec(shape, lambda b, _n=n: (0,) * _n)

    return pl.pallas_call(
        kern,
        out_shape=jax.ShapeDtypeStruct((B, N, C), x.dtype),
        grid=(B,),
        in_specs=[pl.BlockSpec((None, N, C), lambda b: (b, 0, 0)),
                  full((1, C)), full((5, 5, C)),
                  full((C, 3 * C)), full((1, 3 * C)),
                  full((3, 3, C)), full((1, 1, C)),
                  full((C, C)), full((1, C)),
                  full((1, C)), full((1, C)), full((5, 5, C)),
                  full((C, Hd + C)), full((Hd, C)), full((1, C)),
                  full((2 * C, 9 * C)), full((1, C)),
                  full((C, 9 * C)), full((1, C)),
                  full((2 * C, C)), full((1, C))],
        out_specs=pl.BlockSpec((None, N, C), lambda b: (b, 0, 0)),
        scratch_shapes=[pltpu.VMEM((H + 4, W + 4, C), jnp.float32),
                        pltpu.VMEM((nW, ws + 2, ws + 2, C), jnp.float32),
                        pltpu.VMEM((H, W, C), jnp.float32),
                        pltpu.VMEM((N + 128, 9 * C), jnp.float32)],
        compiler_params=pltpu.CompilerParams(dimension_semantics=("parallel",)),
    )(x,
      rn1_w.reshape(1, C), att_os_w,
      wqkv, bqkv,
      att_w3, att_b3,
      att_wp, att_bp,
      gamma1.reshape(1, C), rn2_w.reshape(1, C), ffn_os_w,
      fkr, ffn_wv, gamma2.reshape(1, C),
      w1c, conv_b1.reshape(1, C), w2c, conv_b2.reshape(1, C),
      conv_w11, conv_b11.reshape(1, C))


# speedup vs baseline: 1.4199x; 1.0693x over previous
import functools

import jax
import jax.numpy as jnp
from jax import lax
from jax.experimental import pallas as pl
from jax.experimental.pallas import tpu as pltpu

_EPS = float(jnp.finfo(jnp.float32).eps)


def _mish(x):
    sp = jnp.maximum(x, 0.0) + jnp.log(1.0 + jnp.exp(-jnp.abs(x)))
    return x * jnp.tanh(sp)


def _rmsnorm(x, w):
    ms = jnp.mean(x * x, axis=-1, keepdims=True)
    return x * lax.rsqrt(ms + _EPS) * w


def _dw5(pad_ref, xn, w, H, W, C):
    pad_ref[...] = jnp.zeros((H + 4, W + 4, C), jnp.float32)
    pad_ref[2:2 + H, 2:2 + W, :] = xn
    acc = jnp.zeros((H, W, C), jnp.float32)
    for kh in range(5):
        for kw in range(5):
            acc += pad_ref[kh:kh + H, kw:kw + W, :] * w[kh, kw]
    return acc


def _fused_kernel(x_ref, rn1_ref, os1_ref, wqkv_ref, bqkv_ref,
                  w3_ref, b3_ref, wp_ref, bp_ref,
                  g1_ref, rn2_ref, os2_ref,
                  fkr_ref, fwv_ref, g2_ref,
                  w1c_ref, b1_ref, w2c_ref, b2_ref, w11_ref, b11_ref,
                  o_ref,
                  pad_ref, vpad_ref, yimg_ref, xpad_ref, zpad_ref,
                  *, H, W, ws, scale, Hd):
    C = o_ref.shape[-1]
    N = H * W
    nWh, nWw = H // ws, W // ws
    nW = nWh * nWw
    NN = ws * ws
    P = 64

    xf = x_ref[...].astype(jnp.float32)

    xn = _rmsnorm(xf, rn1_ref[...])
    conv1 = _dw5(pad_ref, xn.reshape(H, W, C), os1_ref[...], H, W, C)

    wins = []
    for wi in range(nWh):
        for wj in range(nWw):
            wins.append(conv1[wi * ws:(wi + 1) * ws,
                              wj * ws:(wj + 1) * ws, :].reshape(1, NN, C))
    xw = jnp.concatenate(wins, axis=0).reshape(nW * NN, C)

    qkv = jnp.dot(xw, wqkv_ref[...],
                  preferred_element_type=jnp.float32) + bqkv_ref[...]
    q = qkv[:, :C]
    k = qkv[:, C:2 * C]
    v = qkv[:, 2 * C:]

    vpad_ref[...] = jnp.zeros((nW, ws + 2, ws + 2, C), jnp.float32)
    vpad_ref[:, 1:ws + 1, 1:ws + 1, :] = v.reshape(nW, ws, ws, C)
    w3 = w3_ref[...]
    lepe = jnp.zeros((nW, ws, ws, C), jnp.float32)
    for kh in range(3):
        for kw in range(3):
            lepe += vpad_ref[:, kh:kh + ws, kw:kw + ws, :] * w3[kh, kw]
    lepe = lepe.reshape(nW, NN, C) + b3_ref[...].reshape(1, 1, C)

    q3 = q.reshape(nW, NN, C)
    k3 = k.reshape(nW, NN, C)
    v3 = v.reshape(nW, NN, C)
    attn = jnp.einsum('gnc,gmc->gnm', q3, k3,
                      preferred_element_type=jnp.float32) * scale
    attn = jax.nn.softmax(attn, axis=-1)
    yw = jnp.einsum('gnm,gmc->gnc', attn, v3,
                    preferred_element_type=jnp.float32) + lepe
    yw = (jnp.dot(yw.reshape(nW * NN, C), wp_ref[...],
                  preferred_element_type=jnp.float32) + bp_ref[...]).reshape(nW, NN, C)

    for wi in range(nWh):
        for wj in range(nWw):
            yimg_ref[wi * ws:(wi + 1) * ws, wj * ws:(wj + 1) * ws, :] = (
                yw[wi * nWw + wj].reshape(ws, ws, C))
    y = yimg_ref[...].reshape(N, C)

    x1 = xf + g1_ref[...] * y
    xn2 = _rmsnorm(x1, rn2_ref[...])
    conv2 = _dw5(pad_ref, xn2.reshape(H, W, C), os2_ref[...], H, W, C)

    a = conv2.reshape(N, C)
    kr = jnp.dot(a, fkr_ref[...], preferred_element_type=jnp.float32)
    kk = jnp.square(jnp.maximum(kr[:, :Hd], 0.0))
    r = jax.nn.sigmoid(kr[:, Hd:])
    kv = jnp.dot(kk, fwv_ref[...], preferred_element_type=jnp.float32)
    x2 = x1 + g2_ref[...] * (r * kv)

    wcol = lax.broadcasted_iota(jnp.int32, (N, C), 0) % W
    mask_l = wcol == 0
    mask_r = wcol == W - 1

    xpad_ref[0:P, :] = jnp.zeros((P, 2 * C), jnp.float32)
    xpad_ref[P + N:P + N + P, :] = jnp.zeros((P, 2 * C), jnp.float32)
    zpad_ref[0:8, :] = jnp.zeros((8, 2 * C), jnp.float32)
    zpad_ref[8 + N:16 + N, :] = jnp.zeros((8, 2 * C), jnp.float32)

    def conv3w(Cin, wr_ref):
        rr = jnp.concatenate(
            [xpad_ref[P + W * (kh - 1):P + W * (kh - 1) + N, 0:Cin]
             for kh in range(3)], axis=1)
        z = jnp.dot(rr, wr_ref[...], preferred_element_type=jnp.float32)
        zpad_ref[8:8 + N, 0:C] = z[:, 0:C]
        zpad_ref[8:8 + N, C:2 * C] = z[:, 2 * C:3 * C]
        acc = z[:, C:2 * C]
        acc = acc + jnp.where(mask_l, 0.0, zpad_ref[7:7 + N, 0:C])
        acc = acc + jnp.where(mask_r, 0.0, zpad_ref[9:9 + N, C:2 * C])
        return acc

    xpad_ref[P:P + N, 0:C] = xf
    xpad_ref[P:P + N, C:2 * C] = x2
    h = _mish(conv3w(2 * C, w1c_ref) + b1_ref[...])
    xpad_ref[P:P + N, 0:C] = h
    h = _mish(conv3w(C, w2c_ref) + b2_ref[...])
    cc = jnp.concatenate([xf, x2], axis=1)
    o2 = jnp.dot(cc, w11_ref[...], preferred_element_type=jnp.float32) + b11_ref[...]
    o_ref[...] = (h + o2).astype(o_ref.dtype)


def kernel(x, rn1_w, rn2_w, gamma1, gamma2, att_os_w, ffn_os_w,
           att_wq, att_wk, att_wv, att_bq, att_bk, att_bv, att_w3, att_b3,
           att_wp, att_bp, ffn_wk, ffn_wv, ffn_wr,
           conv_w1, conv_b1, conv_w2, conv_b2, conv_w11, conv_b11):
    B, N, C = x.shape
    H = W = 32
    ws = 8
    nW = (H // ws) * (W // ws)
    Hd = ffn_wk.shape[1]
    scale = C ** (-0.5)

    wqkv = jnp.concatenate([att_wq, att_wk, att_wv], axis=1)
    bqkv = jnp.concatenate([att_bq, att_bk, att_bv], axis=1)
    fkr = jnp.concatenate([ffn_wk, ffn_wr], axis=1)
    w1c = conv_w1.transpose(0, 2, 1, 3).reshape(3 * 2 * C, 3 * C)
    w2c = conv_w2.transpose(0, 2, 1, 3).reshape(3 * C, 3 * C)

    kern = functools.partial(_fused_kernel, H=H, W=W, ws=ws, scale=scale, Hd=Hd)

    def full(shape):
        n = len(shape)
        return pl.BlockSpec(shape, lambda b, _n=n: (0,) * _n)

    return pl.pallas_call(
        kern,
        out_shape=jax.ShapeDtypeStruct((B, N, C), x.dtype),
        grid=(B,),
        in_specs=[pl.BlockSpec((None, N, C), lambda b: (b, 0, 0)),
                  full((1, C)), full((5, 5, C)),
                  full((C, 3 * C)), full((1, 3 * C)),
                  full((3, 3, C)), full((1, 1, C)),
                  full((C, C)), full((1, C)),
                  full((1, C)), full((1, C)), full((5, 5, C)),
                  full((C, Hd + C)), full((Hd, C)), full((1, C)),
                  full((6 * C, 3 * C)), full((1, C)),
                  full((3 * C, 3 * C)), full((1, C)),
                  full((2 * C, C)), full((1, C))],
        out_specs=pl.BlockSpec((None, N, C), lambda b: (b, 0, 0)),
        scratch_shapes=[pltpu.VMEM((H + 4, W + 4, C), jnp.float32),
                        pltpu.VMEM((nW, ws + 2, ws + 2, C), jnp.float32),
                        pltpu.VMEM((H, W, C), jnp.float32),
                        pltpu.VMEM((N + 128, 2 * C), jnp.float32),
                        pltpu.VMEM((N + 16, 2 * C), jnp.float32)],
        compiler_params=pltpu.CompilerParams(dimension_semantics=("parallel",)),
    )(x,
      rn1_w.reshape(1, C), att_os_w,
      wqkv, bqkv,
      att_w3, att_b3,
      att_wp, att_bp,
      gamma1.reshape(1, C), rn2_w.reshape(1, C), ffn_os_w,
      fkr, ffn_wv, gamma2.reshape(1, C),
      w1c, conv_b1.reshape(1, C), w2c, conv_b2.reshape(1, C),
      conv_w11, conv_b11.reshape(1, C))


# speedup vs baseline: 1.4828x; 1.0443x over previous
import functools

import jax
import jax.numpy as jnp
from jax import lax
from jax.experimental import pallas as pl
from jax.experimental.pallas import tpu as pltpu

_EPS = float(jnp.finfo(jnp.float32).eps)


def _mish(x):
    sp = jnp.maximum(x, 0.0) + jnp.log(1.0 + jnp.exp(-jnp.abs(x)))
    return x * jnp.tanh(sp)


def _rmsnorm(x, w):
    ms = jnp.mean(x * x, axis=-1, keepdims=True)
    return x * lax.rsqrt(ms + _EPS) * w


def _dw5(pad_ref, xn, w, H, W, C):
    pad_ref[2:2 + H, 2:2 + W, :] = xn
    acc = jnp.zeros((H, W, C), jnp.float32)
    for kh in range(5):
        for kw in range(5):
            acc += pad_ref[kh:kh + H, kw:kw + W, :] * w[kh, kw]
    return acc


def _one_image(xf, rn1_ref, os1_ref, wqkv_ref, bqkv_ref,
               w3_ref, b3_ref, wp_ref, bp_ref,
               g1_ref, rn2_ref, os2_ref,
               fkr_ref, fwv_ref, g2_ref,
               w1c_ref, b1_ref, w2c_ref, b2_ref, w11_ref, b11_ref,
               pad_ref, vpad_ref, yimg_ref, xpad_ref, zpad_ref,
               mask_l, mask_r, H, W, ws, scale, Hd):
    C = xf.shape[-1]
    N = H * W
    nWh, nWw = H // ws, W // ws
    nW = nWh * nWw
    NN = ws * ws
    P = 64

    pad_ref[...] = jnp.zeros((H + 4, W + 4, C), jnp.float32)
    xn = _rmsnorm(xf, rn1_ref[...])
    conv1 = _dw5(pad_ref, xn.reshape(H, W, C), os1_ref[...], H, W, C)

    wins = []
    for wi in range(nWh):
        for wj in range(nWw):
            wins.append(conv1[wi * ws:(wi + 1) * ws,
                              wj * ws:(wj + 1) * ws, :].reshape(1, NN, C))
    xw = jnp.concatenate(wins, axis=0).reshape(nW * NN, C)

    qkv = jnp.dot(xw, wqkv_ref[...],
                  preferred_element_type=jnp.float32) + bqkv_ref[...]
    q = qkv[:, :C]
    k = qkv[:, C:2 * C]
    v = qkv[:, 2 * C:]

    vpad_ref[...] = jnp.zeros((nW, ws + 2, ws + 2, C), jnp.float32)
    vpad_ref[:, 1:ws + 1, 1:ws + 1, :] = v.reshape(nW, ws, ws, C)
    w3 = w3_ref[...]
    lepe = jnp.zeros((nW, ws, ws, C), jnp.float32)
    for kh in range(3):
        for kw in range(3):
            lepe += vpad_ref[:, kh:kh + ws, kw:kw + ws, :] * w3[kh, kw]
    lepe = lepe.reshape(nW, NN, C) + b3_ref[...].reshape(1, 1, C)

    q3 = q.reshape(nW, NN, C)
    k3 = k.reshape(nW, NN, C)
    v3 = v.reshape(nW, NN, C)
    attn = jnp.einsum('gnc,gmc->gnm', q3, k3,
                      preferred_element_type=jnp.float32) * scale
    attn = jax.nn.softmax(attn, axis=-1)
    yw = jnp.einsum('gnm,gmc->gnc', attn, v3,
                    preferred_element_type=jnp.float32) + lepe
    yw = (jnp.dot(yw.reshape(nW * NN, C), wp_ref[...],
                  preferred_element_type=jnp.float32) + bp_ref[...]).reshape(nW, NN, C)

    for wi in range(nWh):
        for wj in range(nWw):
            yimg_ref[wi * ws:(wi + 1) * ws, wj * ws:(wj + 1) * ws, :] = (
                yw[wi * nWw + wj].reshape(ws, ws, C))
    y = yimg_ref[...].reshape(N, C)

    x1 = xf + g1_ref[...] * y
    xn2 = _rmsnorm(x1, rn2_ref[...])
    conv2 = _dw5(pad_ref, xn2.reshape(H, W, C), os2_ref[...], H, W, C)

    a = conv2.reshape(N, C)
    kr = jnp.dot(a, fkr_ref[...], preferred_element_type=jnp.float32)
    kk = jnp.square(jnp.maximum(kr[:, :Hd], 0.0))
    r = jax.nn.sigmoid(kr[:, Hd:])
    kv = jnp.dot(kk, fwv_ref[...], preferred_element_type=jnp.float32)
    x2 = x1 + g2_ref[...] * (r * kv)

    xpad_ref[0:P, :] = jnp.zeros((P, 2 * C), jnp.float32)
    xpad_ref[P + N:P + N + P, :] = jnp.zeros((P, 2 * C), jnp.float32)
    zpad_ref[0:8, :] = jnp.zeros((8, 2 * C), jnp.float32)
    zpad_ref[8 + N:16 + N, :] = jnp.zeros((8, 2 * C), jnp.float32)

    def conv3w(Cin, wr_ref):
        rr = jnp.concatenate(
            [xpad_ref[P + W * (kh - 1):P + W * (kh - 1) + N, 0:Cin]
             for kh in range(3)], axis=1)
        z = jnp.dot(rr, wr_ref[...], preferred_element_type=jnp.float32)
        zpad_ref[8:8 + N, 0:C] = z[:, 0:C]
        zpad_ref[8:8 + N, C:2 * C] = z[:, 2 * C:3 * C]
        acc = z[:, C:2 * C]
        acc = acc + jnp.where(mask_l, 0.0, zpad_ref[7:7 + N, 0:C])
        acc = acc + jnp.where(mask_r, 0.0, zpad_ref[9:9 + N, C:2 * C])
        return acc

    xpad_ref[P:P + N, 0:C] = xf
    xpad_ref[P:P + N, C:2 * C] = x2
    h = _mish(conv3w(2 * C, w1c_ref) + b1_ref[...])
    xpad_ref[P:P + N, 0:C] = h
    h = _mish(conv3w(C, w2c_ref) + b2_ref[...])
    cc = jnp.concatenate([xf, x2], axis=1)
    o2 = jnp.dot(cc, w11_ref[...], preferred_element_type=jnp.float32) + b11_ref[...]
    return h + o2


def _fused_kernel(x_ref, rn1_ref, os1_ref, wqkv_ref, bqkv_ref,
                  w3_ref, b3_ref, wp_ref, bp_ref,
                  g1_ref, rn2_ref, os2_ref,
                  fkr_ref, fwv_ref, g2_ref,
                  w1c_ref, b1_ref, w2c_ref, b2_ref, w11_ref, b11_ref,
                  o_ref,
                  pad0, pad1, vpad0, vpad1, yimg0, yimg1,
                  xpad0, xpad1, zpad0, zpad1,
                  *, H, W, ws, scale, Hd):
    C = o_ref.shape[-1]
    N = H * W

    wcol = lax.broadcasted_iota(jnp.int32, (N, C), 0) % W
    mask_l = wcol == 0
    mask_r = wcol == W - 1

    wargs = (rn1_ref, os1_ref, wqkv_ref, bqkv_ref, w3_ref, b3_ref,
             wp_ref, bp_ref, g1_ref, rn2_ref, os2_ref, fkr_ref, fwv_ref,
             g2_ref, w1c_ref, b1_ref, w2c_ref, b2_ref, w11_ref, b11_ref)

    out0 = _one_image(x_ref[0].astype(jnp.float32), *wargs,
                      pad0, vpad0, yimg0, xpad0, zpad0,
                      mask_l, mask_r, H, W, ws, scale, Hd)
    out1 = _one_image(x_ref[1].astype(jnp.float32), *wargs,
                      pad1, vpad1, yimg1, xpad1, zpad1,
                      mask_l, mask_r, H, W, ws, scale, Hd)
    o_ref[0] = out0.astype(o_ref.dtype)
    o_ref[1] = out1.astype(o_ref.dtype)


def kernel(x, rn1_w, rn2_w, gamma1, gamma2, att_os_w, ffn_os_w,
           att_wq, att_wk, att_wv, att_bq, att_bk, att_bv, att_w3, att_b3,
           att_wp, att_bp, ffn_wk, ffn_wv, ffn_wr,
           conv_w1, conv_b1, conv_w2, conv_b2, conv_w11, conv_b11):
    B, N, C = x.shape
    H = W = 32
    ws = 8
    nW = (H // ws) * (W // ws)
    Hd = ffn_wk.shape[1]
    scale = C ** (-0.5)

    wqkv = jnp.concatenate([att_wq, att_wk, att_wv], axis=1)
    bqkv = jnp.concatenate([att_bq, att_bk, att_bv], axis=1)
    fkr = jnp.concatenate([ffn_wk, ffn_wr], axis=1)
    w1c = conv_w1.transpose(0, 2, 1, 3).reshape(3 * 2 * C, 3 * C)
    w2c = conv_w2.transpose(0, 2, 1, 3).reshape(3 * C, 3 * C)

    kern = functools.partial(_fused_kernel, H=H, W=W, ws=ws, scale=scale, Hd=Hd)

    def full(shape):
        n = len(shape)
        return pl.BlockSpec(shape, lambda b, _n=n: (0,) * _n)

    f32 = jnp.float32
    return pl.pallas_call(
        kern,
        out_shape=jax.ShapeDtypeStruct((B, N, C), x.dtype),
        grid=(B // 2,),
        in_specs=[pl.BlockSpec((2, N, C), lambda b: (b, 0, 0)),
                  full((1, C)), full((5, 5, C)),
                  full((C, 3 * C)), full((1, 3 * C)),
                  full((3, 3, C)), full((1, 1, C)),
                  full((C, C)), full((1, C)),
                  full((1, C)), full((1, C)), full((5, 5, C)),
                  full((C, Hd + C)), full((Hd, C)), full((1, C)),
                  full((6 * C, 3 * C)), full((1, C)),
                  full((3 * C, 3 * C)), full((1, C)),
                  full((2 * C, C)), full((1, C))],
        out_specs=pl.BlockSpec((2, N, C), lambda b: (b, 0, 0)),
        scratch_shapes=[pltpu.VMEM((H + 4, W + 4, C), f32),
                        pltpu.VMEM((H + 4, W + 4, C), f32),
                        pltpu.VMEM((nW, ws + 2, ws + 2, C), f32),
                        pltpu.VMEM((nW, ws + 2, ws + 2, C), f32),
                        pltpu.VMEM((H, W, C), f32),
                        pltpu.VMEM((H, W, C), f32),
                        pltpu.VMEM((N + 128, 2 * C), f32),
                        pltpu.VMEM((N + 128, 2 * C), f32),
                        pltpu.VMEM((N + 16, 2 * C), f32),
                        pltpu.VMEM((N + 16, 2 * C), f32)],
        compiler_params=pltpu.CompilerParams(dimension_semantics=("parallel",)),
    )(x,
      rn1_w.reshape(1, C), att_os_w,
      wqkv, bqkv,
      att_w3, att_b3,
      att_wp, att_bp,
      gamma1.reshape(1, C), rn2_w.reshape(1, C), ffn_os_w,
      fkr, ffn_wv, gamma2.reshape(1, C),
      w1c, conv_b1.reshape(1, C), w2c, conv_b2.reshape(1, C),
      conv_w11, conv_b11.reshape(1, C))
